# Initial kernel scaffold; baseline (speedup 1.0000x reference)
import jax
import jax.numpy as jnp
from jax.experimental import pallas as pl


def kernel(pred_confs, pred_rboxes, true_rboxes, predicts, pos_indicator, targets, predict_lengths, target_lengths):
    raise NotImplementedError("write your pallas kernel here")


# speedup vs baseline: 18.6667x; 18.6667x over previous
import functools

import jax
import jax.numpy as jnp
from jax.experimental import pallas as pl

EPS = 1e-07
NEG = -1e30
THETA_COEF = 10.0
HARD_NEG = 512
RAND_NEG = 512
HARD_POS = 128
RAND_POS = 128


def _lae(a, b):
    m = jnp.maximum(a, b)
    return m + jnp.log(1.0 + jnp.exp(-jnp.abs(a - b)))


def _lae3(a, b, c):
    return _lae(_lae(a, b), c)


def _kth_largest_bits(bits, maskb, k):
    def step(_, lh):
        lo, hi = lh
        mid = (lo + hi + 1) // 2
        cnt = jnp.sum(jnp.where(maskb & (bits >= mid), 1.0, 0.0))
        take = cnt >= k
        return (jnp.where(take, mid, lo), jnp.where(take, hi, mid - 1))

    lo, _ = jax.lax.fori_loop(
        0, 31, step, (jnp.int32(0), jnp.int32(0x3F800000)))
    return lo


def _fots_body(conf_ref, pos_ref, pd_ref, td_ref, lpT_ref, blankT_ref,
               tgtT_ref, tlen_ref, plen_ref, out_ref):
    conf = conf_ref[0]
    posf = pos_ref[0]
    posb = posf > 0.5
    negb = jnp.logical_not(posb)

    n_pos = jnp.sum(posf)
    n_neg = 16384.0 - n_pos

    pc = jnp.clip(conf, EPS, 1.0 - EPS)
    bce1 = -jnp.log(pc)
    bce0 = -jnp.log(1.0 - pc)

    pos_loss = jnp.sum(jnp.where(posb, bce1, 0.0))
    bits = jax.lax.bitcast_convert_type(conf, jnp.int32)
    k_neg = jnp.minimum(n_neg, float(HARD_NEG + RAND_NEG))
    tnb = _kth_largest_bits(bits, negb, k_neg)
    gtn = negb & (bits > tnb)
    g_neg = jnp.sum(jnp.where(gtn, 1.0, 0.0))
    neg_sum = jnp.sum(jnp.where(gtn, bce0, 0.0))
    tval = jax.lax.bitcast_convert_type(tnb, jnp.float32)
    tclip = jnp.clip(tval, EPS, 1.0 - EPS)
    neg_sum = neg_sum + (k_neg - g_neg) * (-jnp.log(1.0 - tclip))
    cls = (pos_loss + neg_sum) / (1.0 + k_neg)

    k_pos = jnp.minimum(n_pos, float(HARD_POS + RAND_POS))
    tpb = _kth_largest_bits(bits, posb, k_pos)
    gtp = posb & (bits > tpb)
    g_pos = jnp.sum(jnp.where(gtp, 1.0, 0.0))
    r_t = k_pos - g_pos
    tieb = posb & (bits == tpb)
    ridx = jax.lax.broadcasted_iota(jnp.int32, (128, 128), 0)
    cidx = jax.lax.broadcasted_iota(jnp.int32, (128, 128), 1)
    idx = ridx * 128 + cidx

    def istep(_, lh):
        lo, hi = lh
        mid = (lo + hi) // 2
        cnt = jnp.sum(jnp.where(tieb & (idx <= mid), 1.0, 0.0))
        ok = cnt >= r_t
        return (jnp.where(ok, lo, mid + 1), jnp.where(ok, hi, mid))

    istar, _ = jax.lax.fori_loop(
        0, 14, istep, (jnp.int32(0), jnp.int32(16383)))
    selb = gtp | (tieb & (idx <= istar) & (r_t > 0.5))

    pd0 = pd_ref[0, 0]; pd1 = pd_ref[0, 1]; pd2 = pd_ref[0, 2]
    pd3 = pd_ref[0, 3]; pa = pd_ref[0, 4]
    td0 = td_ref[0, 0]; td1 = td_ref[0, 1]; td2 = td_ref[0, 2]
    td3 = td_ref[0, 3]; ta = td_ref[0, 4]
    area_p = (pd0 + pd2) * (pd1 + pd3)
    area_t = (td0 + td2) * (td1 + td3)
    ih = jnp.minimum(pd0, td0) + jnp.minimum(pd2, td2)
    iw = jnp.minimum(pd1, td1) + jnp.minimum(pd3, td3)
    inter = ih * iw
    union = area_p + area_t - inter
    iou = (inter + 1.0) / (union + 1.0)
    loc = -jnp.log(iou)
    orient = 1.0 - jnp.cos(pa - ta)
    reg = jnp.sum(jnp.where(selb, loc + THETA_COEF * orient, 0.0)) / k_pos

    tgtT = tgtT_ref[0]
    tlen_row = tlen_ref[0]
    plen_row = plen_ref[0]
    blankT = blankT_ref[0]

    cio = jax.lax.broadcasted_iota(jnp.int32, (16, 64), 1)
    g_list = []
    for n in range(8):
        oh = (tgtT[:, n:n + 1] == cio).astype(jnp.float32)
        g_list.append(jnp.dot(oh, lpT_ref[0, n],
                              precision=jax.lax.Precision.HIGHEST,
                              preferred_element_type=jnp.float32))

    negrow = jnp.full((1, 8), NEG, dtype=jnp.float32)
    tprev = jnp.concatenate([tgtT[:1], tgtT[:-1]], axis=0)
    allow_iota = jax.lax.broadcasted_iota(jnp.int32, (16, 8), 0)
    allowed = (allow_iota >= 1) & (tgtT != tprev)

    def g_at(t):
        cols = [g_list[n][:, t:t + 1] for n in range(8)]
        return jnp.concatenate(cols, axis=1)

    s_iota16 = jax.lax.broadcasted_iota(jnp.int32, (16, 8), 0)
    s_iota17 = jax.lax.broadcasted_iota(jnp.int32, (17, 8), 0)

    blank0 = blankT[0:1, :]
    a_b = jnp.where(s_iota17 == 0, jnp.broadcast_to(blank0, (17, 8)), NEG)
    g0 = g_at(0)
    a_s = jnp.where((s_iota16 == 0) & (tlen_row > 0), g0, NEG)

    for t in range(1, 64):
        blank_t = blankT[t:t + 1, :]
        g_t = g_at(t)
        a_s_shift = jnp.concatenate([negrow, a_s[:-1]], axis=0)
        skip = jnp.where(allowed, a_s_shift, NEG)
        new_a_s = _lae3(a_s, a_b[:16], skip) + g_t
        a_s_up = jnp.concatenate([negrow, a_s], axis=0)
        new_a_b = _lae(a_b, a_s_up) + blank_t
        tmask = t < plen_row
        a_b = jnp.where(tmask, new_a_b, a_b)
        a_s = jnp.where(tmask, new_a_s, a_s)

    oh_last = s_iota17 == tlen_row
    a_last = jnp.sum(jnp.where(oh_last, a_b, 0.0), axis=0, keepdims=True)
    oh_prev = (s_iota16 == tlen_row - 1) & (tlen_row >= 1)
    a_prev_s = jnp.sum(jnp.where(oh_prev, a_s, 0.0), axis=0, keepdims=True)
    a_prev = jnp.where(tlen_row >= 1, a_prev_s, a_last)
    ll = _lae(a_last, a_prev)
    closs = -ll
    closs = jnp.where(closs < 1e29, closs, 0.0)
    closs = closs / jnp.maximum(tlen_row.astype(jnp.float32), 1.0)
    ctc = jnp.sum(closs) / 8.0

    total = cls + reg + ctc
    out_ref[...] = jnp.broadcast_to(
        jnp.reshape(total, (1, 1, 1)), (1, 1, 128))


def kernel(pred_confs, pred_rboxes, true_rboxes, predicts, pos_indicator,
           targets, predict_lengths, target_lengths):
    B, H, W = pos_indicator.shape
    T, NT, C = predicts.shape[1], predicts.shape[2], predicts.shape[3]
    conf = pred_confs.reshape(B, H, W)
    posf = pos_indicator.astype(jnp.float32)
    pd = jnp.transpose(pred_rboxes, (0, 3, 1, 2))
    td = jnp.transpose(true_rboxes, (0, 3, 1, 2))
    lpT = jnp.transpose(predicts, (0, 2, 3, 1))
    blankT = predicts[:, :, :, 0]
    tgtT = jnp.transpose(targets, (0, 2, 1)).astype(jnp.int32)
    tlen = target_lengths.reshape(B, 1, NT).astype(jnp.int32)
    plen = predict_lengths.reshape(B, 1, NT).astype(jnp.int32)

    out = pl.pallas_call(
        _fots_body,
        grid=(B,),
        in_specs=[
            pl.BlockSpec((1, H, W), lambda b: (b, 0, 0)),
            pl.BlockSpec((1, H, W), lambda b: (b, 0, 0)),
            pl.BlockSpec((1, 5, H, W), lambda b: (b, 0, 0, 0)),
            pl.BlockSpec((1, 5, H, W), lambda b: (b, 0, 0, 0)),
            pl.BlockSpec((1, NT, C, T), lambda b: (b, 0, 0, 0)),
            pl.BlockSpec((1, T, NT), lambda b: (b, 0, 0)),
            pl.BlockSpec((1, 16, NT), lambda b: (b, 0, 0)),
            pl.BlockSpec((1, 1, NT), lambda b: (b, 0, 0)),
            pl.BlockSpec((1, 1, NT), lambda b: (b, 0, 0)),
        ],
        out_specs=pl.BlockSpec((1, 1, 128), lambda b: (b, 0, 0)),
        out_shape=jax.ShapeDtypeStruct((B, 1, 128), jnp.float32),
    )(conf, posf, pd, td, lpT, blankT, tgtT, tlen, plen)
    return out[:, 0, 0]


# speedup vs baseline: 38.1623x; 2.0444x over previous
import jax
import jax.numpy as jnp
from jax.experimental import pallas as pl

EPS = 1e-07
NEG = -1e30
THETA_COEF = 10.0
N_NEG_SEL = 1024.0
N_POS_SEL = 256.0
HIGH = jax.lax.Precision.HIGHEST


def _lae(a, b):
    m = jnp.maximum(a, b)
    return m + jnp.log(1.0 + jnp.exp(-jnp.abs(a - b)))


def _bsum(x):
    return jnp.sum(jnp.sum(x, axis=2), axis=1, keepdims=True)


def _fots_body(conf_ref, pos_ref, pd_ref, td_ref, lpn_ref, tgt_ref,
               tl2_ref, pl256_ref, tlen_ref, out_ref):
    conf = conf_ref[...]
    posf = pos_ref[...]
    posb = posf > 0.5
    negb = jnp.logical_not(posb)

    n_pos = _bsum(posf)
    n_neg = 16384.0 - n_pos

    pc = jnp.clip(conf, EPS, 1.0 - EPS)
    bce1 = -jnp.log(pc)
    bce0 = -jnp.log(1.0 - pc)

    pos_loss = _bsum(jnp.where(posb, bce1, 0.0))
    bits = jax.lax.bitcast_convert_type(conf, jnp.int32)
    k_neg = jnp.minimum(n_neg, N_NEG_SEL)
    k_pos = jnp.minimum(n_pos, N_POS_SEL)

    def search(_, st):
        lon, hin, lop, hip = st
        midn = (lon + hin + 1) // 2
        midp = (lop + hip + 1) // 2
        cn = _bsum(jnp.where(negb & (bits >= midn[:, :, None]), 1.0, 0.0))
        cp = _bsum(jnp.where(posb & (bits >= midp[:, :, None]), 1.0, 0.0))
        tn = cn >= k_neg
        tp = cp >= k_pos
        return (jnp.where(tn, midn, lon), jnp.where(tn, hin, midn - 1),
                jnp.where(tp, midp, lop), jnp.where(tp, hip, midp - 1))

    zero4 = jnp.zeros((4, 1), jnp.int32)
    top4 = jnp.full((4, 1), 0x3F800000, jnp.int32)
    lon, _, lop, _ = jax.lax.fori_loop(
        0, 31, search, (zero4, top4, zero4, top4))

    gtn = negb & (bits > lon[:, :, None])
    g_neg = _bsum(jnp.where(gtn, 1.0, 0.0))
    neg_sum = _bsum(jnp.where(gtn, bce0, 0.0))
    tval = jax.lax.bitcast_convert_type(lon, jnp.float32)
    tclip = jnp.clip(tval, EPS, 1.0 - EPS)
    neg_sum = neg_sum + (k_neg - g_neg) * (-jnp.log(1.0 - tclip))
    cls = (pos_loss + neg_sum) / (1.0 + k_neg)

    gtp = posb & (bits > lop[:, :, None])
    g_pos = _bsum(jnp.where(gtp, 1.0, 0.0))
    r_t = k_pos - g_pos
    tieb = posb & (bits == lop[:, :, None])
    idx = (jax.lax.broadcasted_iota(jnp.int32, (4, 128, 128), 1) * 128
           + jax.lax.broadcasted_iota(jnp.int32, (4, 128, 128), 2))

    def isearch(_, st):
        lo, hi = st
        mid = (lo + hi) // 2
        cnt = _bsum(jnp.where(tieb & (idx <= mid[:, :, None]), 1.0, 0.0))
        ok = cnt >= r_t
        return (jnp.where(ok, lo, mid + 1), jnp.where(ok, hi, mid))

    istar, _ = jax.lax.fori_loop(
        0, 14, isearch, (zero4, jnp.full((4, 1), 16383, jnp.int32)))
    selb = gtp | (tieb & (idx <= istar[:, :, None]) & (r_t[:, :, None] > 0.5))

    pd0 = pd_ref[:, 0]; pd1 = pd_ref[:, 1]; pd2 = pd_ref[:, 2]
    pd3 = pd_ref[:, 3]; pa = pd_ref[:, 4]
    td0 = td_ref[:, 0]; td1 = td_ref[:, 1]; td2 = td_ref[:, 2]
    td3 = td_ref[:, 3]; ta = td_ref[:, 4]
    area_p = (pd0 + pd2) * (pd1 + pd3)
    area_t = (td0 + td2) * (td1 + td3)
    inter = ((jnp.minimum(pd0, td0) + jnp.minimum(pd2, td2))
             * (jnp.minimum(pd1, td1) + jnp.minimum(pd3, td3)))
    union = area_p + area_t - inter
    iou = (inter + 1.0) / (union + 1.0)
    term = -jnp.log(iou) + THETA_COEF * (1.0 - jnp.cos(pa - ta))
    reg = _bsum(jnp.where(selb, term, 0.0)) / k_pos

    tgt_all = tgt_ref[...]
    tl2 = tl2_ref[...]
    pl256 = pl256_ref[...]
    tlen = tlen_ref[...]

    ri = jax.lax.broadcasted_iota(jnp.int32, (64, 256), 0)
    qi = jax.lax.broadcasted_iota(jnp.int32, (64, 256), 1)
    mbd = (((ri // 16) == (qi // 64)) & ((qi % 64) % 2 == 1)
           & (((qi % 64) - 1) // 2 == (ri % 16))).astype(jnp.float32)
    extl_f = jnp.dot(tgt_all, mbd, precision=HIGH,
                     preferred_element_type=jnp.float32)
    extl = extl_f.astype(jnp.int32)

    cr = jax.lax.broadcasted_iota(jnp.int32, (256, 256), 0)
    cq = jax.lax.broadcasted_iota(jnp.int32, (256, 256), 1)
    same_b = (cr // 64) == (cq // 64)
    c_row = cr % 64
    emits = []
    for n in range(8):
        ohe = (same_b & (extl[n:n + 1, :] == c_row)).astype(jnp.float32)
        emits.append(jnp.dot(lpn_ref[n], ohe, precision=HIGH,
                             preferred_element_type=jnp.float32))

    l_in = jax.lax.broadcasted_iota(jnp.int32, (8, 256), 1) % 64
    extl_s2 = jnp.concatenate([extl[:, -2:], extl[:, :-2]], axis=1)
    allowed = (l_in >= 2) & (extl != extl_s2)
    negc1 = jnp.full((8, 1), NEG, jnp.float32)
    negc2 = jnp.full((8, 2), NEG, jnp.float32)

    def emit_at(t):
        return jnp.concatenate([emits[n][t:t + 1, :] for n in range(8)],
                               axis=0)

    e0 = emit_at(0)
    alpha = jnp.where((l_in == 0) | ((l_in == 1) & (tl2 > 0)), e0, NEG)

    for t in range(1, 64):
        e_t = emit_at(t)
        s1 = jnp.concatenate([negc1, alpha[:, :-1]], axis=1)
        s1 = jnp.where(l_in >= 1, s1, NEG)
        s2 = jnp.concatenate([negc2, alpha[:, :-2]], axis=1)
        s2 = jnp.where(allowed, s2, NEG)
        new = _lae(alpha, _lae(s1, s2)) + e_t
        alpha = jnp.where(t < pl256, new, alpha)

    ohl = l_in == jnp.clip(tl2, 0, 32)
    ohp = l_in == jnp.clip(tl2 - 1, 0, 32)
    vall = jnp.where(ohl, alpha, 0.0)
    valp = jnp.where(ohp, alpha, 0.0)
    last_cols = [jnp.sum(vall[:, b * 64:(b + 1) * 64], axis=1, keepdims=True)
                 for b in range(4)]
    prev_cols = [jnp.sum(valp[:, b * 64:(b + 1) * 64], axis=1, keepdims=True)
                 for b in range(4)]
    a_last = jnp.concatenate(last_cols, axis=1)
    a_prev = jnp.concatenate(prev_cols, axis=1)
    ll = _lae(a_last, a_prev)
    closs = -ll
    closs = jnp.where(closs < 1e29, closs, 0.0)
    closs = closs / jnp.maximum(tlen.astype(jnp.float32), 1.0)
    ctc_row = jnp.sum(closs, axis=0, keepdims=True) / 8.0

    d0 = jax.lax.broadcasted_iota(jnp.int32, (4, 4), 0)
    d1 = jax.lax.broadcasted_iota(jnp.int32, (4, 4), 1)
    ctc_sq = jnp.where(d0 == d1, jnp.broadcast_to(ctc_row, (4, 4)), 0.0)
    ctc = jnp.sum(ctc_sq, axis=1, keepdims=True)

    total = cls + reg + ctc
    out_ref[...] = jnp.broadcast_to(total, (4, 128))


def kernel(pred_confs, pred_rboxes, true_rboxes, predicts, pos_indicator,
           targets, predict_lengths, target_lengths):
    B, H, W = pos_indicator.shape
    T, NT, C = predicts.shape[1], predicts.shape[2], predicts.shape[3]
    S = targets.shape[2]
    conf = pred_confs.reshape(B, H, W)
    posf = pos_indicator.astype(jnp.float32)
    pd = jnp.transpose(pred_rboxes, (0, 3, 1, 2))
    td = jnp.transpose(true_rboxes, (0, 3, 1, 2))
    lpn = jnp.transpose(predicts, (2, 1, 0, 3)).reshape(NT, T, B * C)
    tgt_all = jnp.transpose(targets, (1, 0, 2)).reshape(
        NT, B * S).astype(jnp.float32)
    tlenT = jnp.transpose(target_lengths, (1, 0)).astype(jnp.int32)
    plenT = jnp.transpose(predict_lengths, (1, 0)).astype(jnp.int32)
    tl2 = jnp.broadcast_to((2 * tlenT)[:, :, None],
                           (NT, B, 64)).reshape(NT, B * 64)
    pl256 = jnp.broadcast_to(plenT[:, :, None],
                             (NT, B, 64)).reshape(NT, B * 64)

    out = pl.pallas_call(
        _fots_body,
        in_specs=[
            pl.BlockSpec((B, H, W), lambda: (0, 0, 0)),
            pl.BlockSpec((B, H, W), lambda: (0, 0, 0)),
            pl.BlockSpec((B, 5, H, W), lambda: (0, 0, 0, 0)),
            pl.BlockSpec((B, 5, H, W), lambda: (0, 0, 0, 0)),
            pl.BlockSpec((NT, T, B * C), lambda: (0, 0, 0)),
            pl.BlockSpec((NT, B * S), lambda: (0, 0)),
            pl.BlockSpec((NT, B * 64), lambda: (0, 0)),
            pl.BlockSpec((NT, B * 64), lambda: (0, 0)),
            pl.BlockSpec((NT, B), lambda: (0, 0)),
        ],
        out_specs=pl.BlockSpec((B, 128), lambda: (0, 0)),
        out_shape=jax.ShapeDtypeStruct((B, 128), jnp.float32),
    )(conf, posf, pd, td, lpn, tgt_all, tl2, pl256, tlenT)
    return out[:, 0]
